# baseline (speedup 1.0000x reference)
import jax
import jax.numpy as jnp
from jax import lax
from jax.experimental import pallas as pl
from jax.experimental.pallas import tpu as pltpu

_N, _D = 4096, 512
_TILE = 512
_GRID = _N // _TILE
_LANES = 128


def _loss_kernel(x0, y0, x1, y1, x2, y2, x3, y3, m0, m1, m2, m3, out_ref):
    lane = lax.broadcasted_iota(jnp.int32, (1, _LANES), 1)
    acc = jnp.zeros((1, _LANES), jnp.float32)
    for k, (x, y, m) in enumerate(
            ((x0, y0, m0), (x1, y1, m1), (x2, y2, m2), (x3, y3, m3))):
        mk = m[...].astype(jnp.float32).reshape(1, _TILE)
        d = x[...] - y[...]
        d2 = d * d
        sv = jnp.dot(mk, d2, preferred_element_type=jnp.float32)
        s = jnp.sum(sv)
        c = jnp.sum(mk)
        acc = acc + jnp.where(lane == k, s, 0.0)
        acc = acc + jnp.where(lane == k + 4, c, 0.0)
    out_ref[0] = acc


def _finish_kernel(part_ref, out_ref):
    total = jnp.float32(0.0)
    for k in range(4):
        s = part_ref[0, 0, k]
        c = part_ref[0, 0, k + 4]
        for g in range(1, _GRID):
            s = s + part_ref[g, 0, k]
            c = c + part_ref[g, 0, k + 4]
        total = total + jnp.where(c > 0, s / jnp.maximum(c * _D, 1.0), 0.0)
    out_ref[0, 0] = total


def _finish(part):
    return pl.pallas_call(
        _finish_kernel,
        out_shape=jax.ShapeDtypeStruct((1, 1), jnp.float32),
        in_specs=[pl.BlockSpec(memory_space=pltpu.MemorySpace.SMEM)],
        out_specs=pl.BlockSpec(memory_space=pltpu.MemorySpace.SMEM),
    )(part)


def _partials(x0, y0, x1, y1, x2, y2, x3, y3, m0, m1, m2, m3):
    xy_spec = pl.BlockSpec((_TILE, _D), lambda g: (g, 0))
    m_spec = pl.BlockSpec((_TILE,), lambda g: (g,))
    return pl.pallas_call(
        _loss_kernel,
        out_shape=jax.ShapeDtypeStruct((_GRID, 1, _LANES), jnp.float32),
        grid=(_GRID,),
        in_specs=[xy_spec] * 8 + [m_spec] * 4,
        out_specs=pl.BlockSpec((1, 1, _LANES), lambda g: (g, 0, 0)),
        compiler_params=pltpu.CompilerParams(
            dimension_semantics=("parallel",),
            vmem_limit_bytes=64 * 1024 * 1024),
    )(x0, y0, x1, y1, x2, y2, x3, y3, m0, m1, m2, m3)


@jax.jit
def kernel(inputs_0, targets_0, masks_0,
           inputs_1, targets_1, masks_1,
           inputs_2, targets_2, masks_2,
           inputs_3, targets_3, masks_3):
    mu = [m.view(jnp.uint8)
          for m in (masks_0, masks_1, masks_2, masks_3)]
    part = _partials(inputs_0, targets_0, inputs_1, targets_1,
                     inputs_2, targets_2, inputs_3, targets_3, *mu)
    return _finish(part).reshape(())


# speedup vs baseline: 1.1658x; 1.1658x over previous
import jax
import jax.numpy as jnp
from jax import lax
from jax.experimental import pallas as pl
from jax.experimental.pallas import tpu as pltpu

_N, _D = 4096, 512
_TILE = 512
_GRID = _N // _TILE
_LANES = 128


def _loss_kernel(x0, y0, x1, y1, x2, y2, x3, y3, m0, m1, m2, m3, out_ref):
    lane = lax.broadcasted_iota(jnp.int32, (1, _LANES), 1)
    acc = jnp.zeros((1, _LANES), jnp.float32)
    for k, (x, y, m) in enumerate(
            ((x0, y0, m0), (x1, y1, m1), (x2, y2, m2), (x3, y3, m3))):
        mk = m[...].reshape(1, _TILE)
        d = x[...] - y[...]
        d2 = d * d
        sv = jnp.dot(mk, d2, preferred_element_type=jnp.float32)
        s = jnp.sum(sv)
        c = jnp.sum(mk)
        acc = acc + jnp.where(lane == k, s, 0.0)
        acc = acc + jnp.where(lane == k + 4, c, 0.0)
    out_ref[0] = acc


def _finish_kernel(part_ref, out_ref):
    total = jnp.float32(0.0)
    for k in range(4):
        s = part_ref[0, 0, k]
        c = part_ref[0, 0, k + 4]
        for g in range(1, _GRID):
            s = s + part_ref[g, 0, k]
            c = c + part_ref[g, 0, k + 4]
        total = total + jnp.where(c > 0, s / jnp.maximum(c * _D, 1.0), 0.0)
    out_ref[0, 0] = total


def _finish(part):
    return pl.pallas_call(
        _finish_kernel,
        out_shape=jax.ShapeDtypeStruct((1, 1), jnp.float32),
        in_specs=[pl.BlockSpec(memory_space=pltpu.MemorySpace.SMEM)],
        out_specs=pl.BlockSpec(memory_space=pltpu.MemorySpace.SMEM),
    )(part)


def _partials(x0, y0, x1, y1, x2, y2, x3, y3, mflat):
    xy_spec = pl.BlockSpec((_TILE, _D), lambda g: (g, 0))
    m_specs = [
        pl.BlockSpec((_TILE,), lambda g, kk=k: (kk * _GRID + g,))
        for k in range(4)
    ]
    return pl.pallas_call(
        _loss_kernel,
        out_shape=jax.ShapeDtypeStruct((_GRID, 1, _LANES), jnp.float32),
        grid=(_GRID,),
        in_specs=[xy_spec] * 8 + m_specs,
        out_specs=pl.BlockSpec((1, 1, _LANES), lambda g: (g, 0, 0)),
        compiler_params=pltpu.CompilerParams(
            dimension_semantics=("parallel",),
            vmem_limit_bytes=64 * 1024 * 1024),
    )(x0, y0, x1, y1, x2, y2, x3, y3, mflat, mflat, mflat, mflat)


@jax.jit
def kernel(inputs_0, targets_0, masks_0,
           inputs_1, targets_1, masks_1,
           inputs_2, targets_2, masks_2,
           inputs_3, targets_3, masks_3):
    mflat = jnp.concatenate(
        (masks_0, masks_1, masks_2, masks_3)).astype(jnp.float32)
    part = _partials(inputs_0, targets_0, inputs_1, targets_1,
                     inputs_2, targets_2, inputs_3, targets_3, mflat)
    return _finish(part).reshape(())
